# baseline (speedup 1.0000x reference)
import jax
import jax.numpy as jnp
from jax import lax
from jax.experimental import pallas as pl
from jax.experimental.pallas import tpu as pltpu
from jax.experimental.pallas import tpu_sc as plsc

NC = 2
NS = 16
L = 16
B = 10000
D = 128
DH = D // NC
E = 320000
BURST = 128
CH = 12
NCHUNK = 14
NPAIR = NCHUNK // 2
NBURST = NCHUNK * CH
WPW = NBURST * BURST
TOT = NS * WPW
PAD_DST = B
ACC_ROWS = 10240
RPW = ACC_ROWS // NS
TPW = B // NS
NPIECE = 5
PJ = TPW // NPIECE
CW = 16
CROWS = ACC_ROWS // CW
CZPW = CROWS // NS


def _sc_body(table2_h, srcI_h, dstI_h, zeros_h,
             out_h,
             srcC0, srcC1, dstC0, dstC1,
             rows0, rows1, rows2, rows3, res_v, cnt_v,
             hist_v, idxv,
             acc_s, cnt_s,
             g0, g1, g2, g3, s0, s1, s2, s3, i0, i1):
    c = lax.axis_index("c")
    s = lax.axis_index("s")

    SRC = (srcC0, srcC1)
    DST = (dstC0, dstC1)
    ROWS = (rows0, rows1, rows2, rows3)
    G = (g0, g1, g2, g3)
    S = (s0, s1, s2, s3)
    I = (i0, i1)

    ones16 = jnp.ones((L,), jnp.float32)
    tab = table2_h.at[c]

    pltpu.sync_copy(zeros_h, acc_s.at[pl.ds(s * RPW, RPW)])
    pltpu.sync_copy(zeros_h.at[pl.ds(0, CZPW), pl.ds(0, CW)],
                    cnt_s.at[pl.ds(s * CZPW, CZPW)])
    pltpu.sync_copy(zeros_h.at[pl.ds(0, CROWS), pl.ds(0, CW)], hist_v)
    for t in range(CROWS // L):
        idxv[pl.ds(t * L, L)] = lax.iota(jnp.int32, L) + (t * L)
    pltpu.sync_copy(srcI_h.at[s].at[pl.ds(0, CH)], srcC0)
    pltpu.sync_copy(dstI_h.at[s].at[pl.ds(0, CH)], dstC0)
    plsc.subcore_barrier()

    def g_start(cp, j, b4):
        pltpu.async_copy(tab.at[SRC[cp].at[j]], ROWS[b4], G[b4])

    def g_wait(cp, j, b4):
        pltpu.make_async_copy(tab.at[SRC[cp].at[j]], ROWS[b4], G[b4]).wait()

    def s_start(cp, j, b4):
        pltpu.async_copy(ROWS[b4], acc_s.at[DST[cp].at[j]], S[b4], add=True)

    def s_wait(cp, j, b4):
        pltpu.make_async_copy(ROWS[b4], acc_s.at[DST[cp].at[j]], S[b4]).wait()

    def count(cp, j):
        for t in range(BURST // L):
            dv = DST[cp][j, pl.ds(t * L, L)]
            row = lax.shift_right_logical(dv, 4)
            lane = lax.bitwise_and(dv, 15)
            plsc.addupdate_scatter(hist_v, [row, lane], ones16)

    def pf_start(off, buf):
        pltpu.async_copy(srcI_h.at[s].at[pl.ds(off, CH)], SRC[buf], I[buf])
        pltpu.async_copy(dstI_h.at[s].at[pl.ds(off, CH)], DST[buf], I[buf])

    def pf_wait(off, buf):
        pltpu.make_async_copy(
            srcI_h.at[s].at[pl.ds(off, CH)], SRC[buf], I[buf]).wait()
        pltpu.make_async_copy(
            dstI_h.at[s].at[pl.ds(off, CH)], DST[buf], I[buf]).wait()

    def chunk_off(ci):
        return jnp.where(ci < NCHUNK, ci * CH, 0)

    def do_chunk(ci, p, first_chunk=False):
        off_next = chunk_off(ci + 1)
        for j in range(CH):
            b4 = j % 4
            if not (first_chunk and j <= 1):
                if j >= 2:
                    s_wait(p, j - 2, (j - 2) % 4)
                elif j == 0:
                    s_wait(1 - p, CH - 2, (CH - 2) % 4)
                else:
                    s_wait(1 - p, CH - 1, (CH - 1) % 4)
            if j == 2:
                pf_start(off_next, 1 - p)
            if j == CH - 2:
                pf_wait(off_next, 1 - p)
                g_start(1 - p, 0, (j + 2) % 4)
            elif j == CH - 1:
                g_start(1 - p, 1, (j + 2) % 4)
            else:
                g_start(p, j + 2, (j + 2) % 4)
            g_wait(p, j, b4)
            s_start(p, j, b4)
            count(p, j)

    g_start(0, 0, 0)
    g_start(0, 1, 1)
    do_chunk(0, 0, first_chunk=True)
    do_chunk(1, 1)

    @pl.loop(1, NPAIR)
    def _(k):
        do_chunk(2 * k, 0)
        do_chunk(2 * k + 1, 1)

    g_wait(0, 0, NBURST % 4)
    g_wait(0, 1, (NBURST + 1) % 4)
    s_wait(1, CH - 2, (NBURST - 2) % 4)
    s_wait(1, CH - 1, (NBURST - 1) % 4)

    pltpu.sync_copy(hist_v, cnt_s.at[idxv], add=True)

    plsc.subcore_barrier()
    @pl.loop(0, NPIECE)
    def _(t):
        base = s * TPW + t * PJ
        crow0 = base // CW
        pltpu.sync_copy(acc_s.at[pl.ds(base, PJ)], res_v)
        pltpu.sync_copy(cnt_s.at[pl.ds(crow0, PJ // CW + 2)], cnt_v)

        @pl.loop(0, PJ)
        def _(r):
            flat = base + r
            rowp = flat // CW - crow0
            lanevec = jnp.broadcast_to(lax.rem(flat, CW), (L,))
            cnt = plsc.load_gather(cnt_v, [jnp.broadcast_to(rowp, (L,)),
                                           lanevec])
            inv = 1.0 / jnp.maximum(cnt, 1.0)
            for q in range(DH // L):
                res_v[r, pl.ds(q * L, L)] = res_v[r, pl.ds(q * L, L)] * inv

        pltpu.sync_copy(res_v,
                        out_h.at[pl.ds(base, PJ), pl.ds(c * DH, DH)])


def _sc_aggregate(table2, src_idx, dst_idx, zeros):
    mesh = plsc.VectorSubcoreMesh(core_axis_name="c", subcore_axis_name="s")
    return pl.kernel(
        _sc_body,
        compiler_params=pltpu.CompilerParams(use_tc_tiling_on_sc=False,
                                             needs_layout_passes=False),
        out_type=jax.ShapeDtypeStruct((B, D), jnp.float32),
        mesh=mesh,
        scratch_types=[
            pltpu.VMEM((CH, BURST), jnp.int32),
            pltpu.VMEM((CH, BURST), jnp.int32),
            pltpu.VMEM((CH, BURST), jnp.int32),
            pltpu.VMEM((CH, BURST), jnp.int32),
            pltpu.VMEM((BURST, DH), jnp.float32),
            pltpu.VMEM((BURST, DH), jnp.float32),
            pltpu.VMEM((BURST, DH), jnp.float32),
            pltpu.VMEM((BURST, DH), jnp.float32),
            pltpu.VMEM((PJ, DH), jnp.float32),
            pltpu.VMEM((PJ // CW + 2, CW), jnp.float32),
            pltpu.VMEM((CROWS, CW), jnp.float32),
            pltpu.VMEM((CROWS,), jnp.int32),
            pltpu.VMEM_SHARED((ACC_ROWS, DH), jnp.float32),
            pltpu.VMEM_SHARED((CROWS, CW), jnp.float32),
            pltpu.SemaphoreType.DMA,
            pltpu.SemaphoreType.DMA,
            pltpu.SemaphoreType.DMA,
            pltpu.SemaphoreType.DMA,
            pltpu.SemaphoreType.DMA,
            pltpu.SemaphoreType.DMA,
            pltpu.SemaphoreType.DMA,
            pltpu.SemaphoreType.DMA,
            pltpu.SemaphoreType.DMA,
            pltpu.SemaphoreType.DMA,
        ],
    )(table2, src_idx, dst_idx, zeros)


def kernel(nodes, edge_index, table):
    dst = edge_index[0]
    src = edge_index[1]
    table2 = table.reshape(B, NC, DH).transpose(1, 0, 2)
    npad = TOT - (E + B)
    dst_idx = jnp.concatenate(
        [dst, jnp.arange(B, dtype=jnp.int32),
         jnp.full((npad,), PAD_DST, jnp.int32)]).reshape(NS, NBURST, BURST)
    src_idx = jnp.concatenate(
        [src, nodes.astype(jnp.int32),
         jnp.zeros((npad,), jnp.int32)]).reshape(NS, NBURST, BURST)
    zeros = jnp.zeros((RPW, DH), jnp.float32)
    return _sc_aggregate(table2, src_idx, dst_idx, zeros)


# speedup vs baseline: 2.4262x; 2.4262x over previous
import jax
import jax.numpy as jnp
from jax import lax
from jax.experimental import pallas as pl
from jax.experimental.pallas import tpu as pltpu
from jax.experimental.pallas import tpu_sc as plsc

NC = 2
NS = 16
L = 16
B = 10000
D = 128
DH = D // NC
E = 320000
BURST = 128
CH = 12
NCHUNK = 14
NPAIR = NCHUNK // 2
NBURST = NCHUNK * CH
WPW = NBURST * BURST
TOT = NS * WPW
PAD_DST = B
ACC_ROWS = 10240
RPW = ACC_ROWS // NS
TPW = B // NS
NPIECE = 5
PJ = TPW // NPIECE
CW = 16
CROWS = ACC_ROWS // CW
CZPW = CROWS // NS


def _sc_body(table_h, idx_h, zeros_h,
             out_h,
             srcC0, srcC1, dstC0, dstC1, rows0, rows1, res_v, cnt_v,
             hist_v, idxv,
             table_s, acc_s, cnt_s,
             g0, g1, s0, s1, i0, i1):
    c = lax.axis_index("c")
    s = lax.axis_index("s")

    SRC = (srcC0, srcC1)
    DST = (dstC0, dstC1)
    ROWS = (rows0, rows1)
    G = (g0, g1)
    S = (s0, s1)
    I = (i0, i1)

    ones16 = jnp.ones((L,), jnp.float32)

    pltpu.sync_copy(table_h.at[pl.ds(s * TPW, TPW), pl.ds(c * DH, DH)],
                    table_s.at[pl.ds(s * TPW, TPW)])
    pltpu.sync_copy(zeros_h, acc_s.at[pl.ds(s * RPW, RPW)])
    pltpu.sync_copy(zeros_h.at[pl.ds(0, CZPW), pl.ds(0, CW)],
                    cnt_s.at[pl.ds(s * CZPW, CZPW)])
    pltpu.sync_copy(zeros_h.at[pl.ds(0, CROWS), pl.ds(0, CW)], hist_v)
    for t in range(CROWS // L):
        idxv[pl.ds(t * L, L)] = lax.iota(jnp.int32, L) + (t * L)
    pltpu.sync_copy(idx_h.at[NS + s].at[pl.ds(0, CH)], srcC0)
    pltpu.sync_copy(idx_h.at[s].at[pl.ds(0, CH)], dstC0)
    plsc.subcore_barrier()

    def g_start(cp, j, bp):
        pltpu.async_copy(table_s.at[SRC[cp].at[j]], ROWS[bp], G[bp])

    def g_wait(cp, j, bp):
        pltpu.make_async_copy(table_s.at[SRC[cp].at[j]], ROWS[bp], G[bp]).wait()

    def s_start(cp, j, bp):
        pltpu.async_copy(ROWS[bp], acc_s.at[DST[cp].at[j]], S[bp], add=True)

    def s_wait(cp, j, bp):
        pltpu.make_async_copy(ROWS[bp], acc_s.at[DST[cp].at[j]], S[bp]).wait()

    def count(cp, j):
        for t in range(BURST // L):
            dv = DST[cp][j, pl.ds(t * L, L)]
            row = lax.shift_right_logical(dv, 4)
            lane = lax.bitwise_and(dv, 15)
            plsc.addupdate_scatter(hist_v, [row, lane], ones16)

    def pf_start(off, buf):
        pltpu.async_copy(idx_h.at[NS + s].at[pl.ds(off, CH)], SRC[buf], I[buf])
        pltpu.async_copy(idx_h.at[s].at[pl.ds(off, CH)], DST[buf], I[buf])

    def pf_wait(off, buf):
        pltpu.make_async_copy(
            idx_h.at[NS + s].at[pl.ds(off, CH)], SRC[buf], I[buf]).wait()
        pltpu.make_async_copy(
            idx_h.at[s].at[pl.ds(off, CH)], DST[buf], I[buf]).wait()

    def chunk_off(ci):
        return jnp.where(ci < NCHUNK, ci * CH, 0)

    def do_chunk(ci, p, first_chunk=False):
        off_next = chunk_off(ci + 1)
        for j in range(CH):
            bp = j % 2
            if first_chunk and j == 0:
                g_start(p, 0, 0)
            else:
                if j > 0:
                    s_wait(p, j - 1, 1 - bp)
                else:
                    s_wait(1 - p, CH - 1, 1 - bp)
            if j == 1:
                pf_start(off_next, 1 - p)
            if j == CH - 1:
                pf_wait(off_next, 1 - p)
                g_start(1 - p, 0, 1 - bp)
            else:
                g_start(p, j + 1, 1 - bp)
            g_wait(p, j, bp)
            s_start(p, j, bp)
            count(p, j)

    do_chunk(0, 0, first_chunk=True)
    do_chunk(1, 1)

    @pl.loop(1, NPAIR)
    def _(k):
        do_chunk(2 * k, 0)
        do_chunk(2 * k + 1, 1)

    g_wait(0, 0, 0)
    s_wait(1, CH - 1, 1)

    pltpu.sync_copy(hist_v, cnt_s.at[idxv], add=True)

    plsc.subcore_barrier()
    @pl.loop(0, NPIECE)
    def _(t):
        base = s * TPW + t * PJ
        crow0 = base // CW
        pltpu.sync_copy(acc_s.at[pl.ds(base, PJ)], res_v)
        pltpu.sync_copy(cnt_s.at[pl.ds(crow0, PJ // CW + 2)], cnt_v)

        @pl.loop(0, PJ)
        def _(r):
            flat = base + r
            rowp = flat // CW - crow0
            lanevec = jnp.broadcast_to(lax.rem(flat, CW), (L,))
            cnt = plsc.load_gather(cnt_v, [jnp.broadcast_to(rowp, (L,)),
                                           lanevec])
            inv = 1.0 / jnp.maximum(cnt, 1.0)
            for q in range(DH // L):
                res_v[r, pl.ds(q * L, L)] = res_v[r, pl.ds(q * L, L)] * inv

        pltpu.sync_copy(res_v,
                        out_h.at[pl.ds(base, PJ), pl.ds(c * DH, DH)])


def _sc_aggregate(table, idx, zeros):
    mesh = plsc.VectorSubcoreMesh(core_axis_name="c", subcore_axis_name="s")
    return pl.kernel(
        _sc_body,
        compiler_params=pltpu.CompilerParams(use_tc_tiling_on_sc=False,
                                             needs_layout_passes=False),
        out_type=jax.ShapeDtypeStruct((B, D), jnp.float32),
        mesh=mesh,
        scratch_types=[
            pltpu.VMEM((CH, BURST), jnp.int32),
            pltpu.VMEM((CH, BURST), jnp.int32),
            pltpu.VMEM((CH, BURST), jnp.int32),
            pltpu.VMEM((CH, BURST), jnp.int32),
            pltpu.VMEM((BURST, DH), jnp.float32),
            pltpu.VMEM((BURST, DH), jnp.float32),
            pltpu.VMEM((PJ, DH), jnp.float32),
            pltpu.VMEM((PJ // CW + 2, CW), jnp.float32),
            pltpu.VMEM((CROWS, CW), jnp.float32),
            pltpu.VMEM((CROWS,), jnp.int32),
            pltpu.VMEM_SHARED((B, DH), jnp.float32),
            pltpu.VMEM_SHARED((ACC_ROWS, DH), jnp.float32),
            pltpu.VMEM_SHARED((CROWS, CW), jnp.float32),
            pltpu.SemaphoreType.DMA,
            pltpu.SemaphoreType.DMA,
            pltpu.SemaphoreType.DMA,
            pltpu.SemaphoreType.DMA,
            pltpu.SemaphoreType.DMA,
            pltpu.SemaphoreType.DMA,
        ],
    )(table, idx, zeros)


def kernel(nodes, edge_index, table):
    dst = edge_index[0]
    src = edge_index[1]
    npad = TOT - (E + B)
    idx = jnp.concatenate(
        [dst, jnp.arange(B, dtype=jnp.int32),
         jnp.full((npad,), PAD_DST, jnp.int32),
         src, nodes.astype(jnp.int32),
         jnp.zeros((npad,), jnp.int32)]).reshape(2 * NS, NBURST, BURST)
    zeros = jnp.zeros((RPW, DH), jnp.float32)
    return _sc_aggregate(table, idx, zeros)


# speedup vs baseline: 2.6944x; 1.1106x over previous
import jax
import jax.numpy as jnp
from jax import lax
from jax.experimental import pallas as pl
from jax.experimental.pallas import tpu as pltpu
from jax.experimental.pallas import tpu_sc as plsc

NC = 2
NS = 16
L = 16
B = 10000
D = 128
DH = D // NC
E = 320000
BURST = 128
CH = 6
NDIR = 26
NEXT = 2
NCHUNK = NDIR + NEXT
NPAIR = NCHUNK // 2
NBURST = NCHUNK * CH
EB = E // BURST
DIRB = NDIR * CH
EREM = E - NS * DIRB * BURST
XTOT = NS * NEXT * CH * BURST
PAD_DST = B
ACC_ROWS = 10240
RPW = ACC_ROWS // NS
TPW = B // NS
NPIECE = 5
PJ = TPW // NPIECE
CW = 16
CROWS = ACC_ROWS // CW
CZPW = CROWS // NS


def _sc_body(table_h, edge_h, extra_h, zeros_h,
             out_h,
             srcC0, srcC1, dstC0, dstC1, rows0, rows1, res_v, cnt_v,
             hist_v, idxv,
             table_s, acc_s, cnt_s,
             g0, g1, s0, s1, i0, i1):
    c = lax.axis_index("c")
    s = lax.axis_index("s")

    SRC = (srcC0, srcC1)
    DST = (dstC0, dstC1)
    ROWS = (rows0, rows1)
    G = (g0, g1)
    S = (s0, s1)
    I = (i0, i1)

    ones16 = jnp.ones((L,), jnp.float32)

    pltpu.sync_copy(table_h.at[pl.ds(s * TPW, TPW), pl.ds(c * DH, DH)],
                    table_s.at[pl.ds(s * TPW, TPW)])
    pltpu.sync_copy(zeros_h, acc_s.at[pl.ds(s * RPW, RPW)])
    pltpu.sync_copy(zeros_h.at[pl.ds(0, CZPW), pl.ds(0, CW)],
                    cnt_s.at[pl.ds(s * CZPW, CZPW)])
    pltpu.sync_copy(zeros_h.at[pl.ds(0, CROWS), pl.ds(0, CW)], hist_v)
    for t in range(CROWS // L):
        idxv[pl.ds(t * L, L)] = lax.iota(jnp.int32, L) + (t * L)
    pltpu.sync_copy(edge_h.at[1].at[pl.ds(s * DIRB, CH)], srcC0)
    pltpu.sync_copy(edge_h.at[0].at[pl.ds(s * DIRB, CH)], dstC0)
    plsc.subcore_barrier()

    def g_start(cp, j, bp):
        pltpu.async_copy(table_s.at[SRC[cp].at[j]], ROWS[bp], G[bp])

    def g_wait(cp, j, bp):
        pltpu.make_async_copy(table_s.at[SRC[cp].at[j]], ROWS[bp], G[bp]).wait()

    def s_start(cp, j, bp):
        pltpu.async_copy(ROWS[bp], acc_s.at[DST[cp].at[j]], S[bp], add=True)

    def s_wait(cp, j, bp):
        pltpu.make_async_copy(ROWS[bp], acc_s.at[DST[cp].at[j]], S[bp]).wait()

    def count(cp, j):
        for t in range(BURST // L):
            dv = DST[cp][j, pl.ds(t * L, L)]
            row = lax.shift_right_logical(dv, 4)
            lane = lax.bitwise_and(dv, 15)
            plsc.addupdate_scatter(hist_v, [row, lane], ones16)

    def pf_start_dir(ci, buf):
        off = s * DIRB + ci * CH
        pltpu.async_copy(edge_h.at[1].at[pl.ds(off, CH)], SRC[buf], I[buf])
        pltpu.async_copy(edge_h.at[0].at[pl.ds(off, CH)], DST[buf], I[buf])

    def pf_wait_dir(ci, buf):
        off = s * DIRB + ci * CH
        pltpu.make_async_copy(
            edge_h.at[1].at[pl.ds(off, CH)], SRC[buf], I[buf]).wait()
        pltpu.make_async_copy(
            edge_h.at[0].at[pl.ds(off, CH)], DST[buf], I[buf]).wait()

    def pf_start_ext(e, buf):
        pltpu.async_copy(extra_h.at[NS + s].at[pl.ds(e * CH, CH)],
                         SRC[buf], I[buf])
        pltpu.async_copy(extra_h.at[s].at[pl.ds(e * CH, CH)],
                         DST[buf], I[buf])

    def pf_wait_ext(e, buf):
        pltpu.make_async_copy(
            extra_h.at[NS + s].at[pl.ds(e * CH, CH)], SRC[buf], I[buf]).wait()
        pltpu.make_async_copy(
            extra_h.at[s].at[pl.ds(e * CH, CH)], DST[buf], I[buf]).wait()

    def do_chunk(p, pf_s, pf_w, first_chunk=False):
        for j in range(CH):
            bp = j % 2
            if first_chunk and j == 0:
                g_start(p, 0, 0)
            else:
                if j > 0:
                    s_wait(p, j - 1, 1 - bp)
                else:
                    s_wait(1 - p, CH - 1, 1 - bp)
            if j == 1:
                pf_s(1 - p)
            if j == CH - 1:
                pf_w(1 - p)
                g_start(1 - p, 0, 1 - bp)
            else:
                g_start(p, j + 1, 1 - bp)
            g_wait(p, j, bp)
            s_start(p, j, bp)
            count(p, j)

    def dirfns(ci):
        return (lambda buf: pf_start_dir(ci, buf),
                lambda buf: pf_wait_dir(ci, buf))

    def extfns(e):
        return (lambda buf: pf_start_ext(e, buf),
                lambda buf: pf_wait_ext(e, buf))

    do_chunk(0, *dirfns(1), first_chunk=True)
    do_chunk(1, *dirfns(2))

    @pl.loop(1, NPAIR - 2)
    def _(k):
        do_chunk(0, *dirfns(2 * k + 1))
        do_chunk(1, *dirfns(2 * k + 2))

    do_chunk(0, *dirfns(NDIR - 1))
    do_chunk(1, *extfns(0))
    do_chunk(0, *extfns(1))
    do_chunk(1, *extfns(0))

    g_wait(0, 0, 0)
    s_wait(1, CH - 1, 1)

    pltpu.sync_copy(hist_v, cnt_s.at[idxv], add=True)

    plsc.subcore_barrier()
    @pl.loop(0, NPIECE)
    def _(t):
        base = s * TPW + t * PJ
        crow0 = base // CW
        pltpu.sync_copy(acc_s.at[pl.ds(base, PJ)], res_v)
        pltpu.sync_copy(cnt_s.at[pl.ds(crow0, PJ // CW + 2)], cnt_v)

        @pl.loop(0, PJ)
        def _(r):
            flat = base + r
            rowp = flat // CW - crow0
            lanevec = jnp.broadcast_to(lax.rem(flat, CW), (L,))
            cnt = plsc.load_gather(cnt_v, [jnp.broadcast_to(rowp, (L,)),
                                           lanevec])
            inv = 1.0 / jnp.maximum(cnt, 1.0)
            for q in range(DH // L):
                res_v[r, pl.ds(q * L, L)] = res_v[r, pl.ds(q * L, L)] * inv

        pltpu.sync_copy(res_v,
                        out_h.at[pl.ds(base, PJ), pl.ds(c * DH, DH)])


def _sc_aggregate(table, edge2, extra, zeros):
    mesh = plsc.VectorSubcoreMesh(core_axis_name="c", subcore_axis_name="s")
    return pl.kernel(
        _sc_body,
        compiler_params=pltpu.CompilerParams(use_tc_tiling_on_sc=False,
                                             needs_layout_passes=False),
        out_type=jax.ShapeDtypeStruct((B, D), jnp.float32),
        mesh=mesh,
        scratch_types=[
            pltpu.VMEM((CH, BURST), jnp.int32),
            pltpu.VMEM((CH, BURST), jnp.int32),
            pltpu.VMEM((CH, BURST), jnp.int32),
            pltpu.VMEM((CH, BURST), jnp.int32),
            pltpu.VMEM((BURST, DH), jnp.float32),
            pltpu.VMEM((BURST, DH), jnp.float32),
            pltpu.VMEM((PJ, DH), jnp.float32),
            pltpu.VMEM((PJ // CW + 2, CW), jnp.float32),
            pltpu.VMEM((CROWS, CW), jnp.float32),
            pltpu.VMEM((CROWS,), jnp.int32),
            pltpu.VMEM_SHARED((B, DH), jnp.float32),
            pltpu.VMEM_SHARED((ACC_ROWS, DH), jnp.float32),
            pltpu.VMEM_SHARED((CROWS, CW), jnp.float32),
            pltpu.SemaphoreType.DMA,
            pltpu.SemaphoreType.DMA,
            pltpu.SemaphoreType.DMA,
            pltpu.SemaphoreType.DMA,
            pltpu.SemaphoreType.DMA,
            pltpu.SemaphoreType.DMA,
        ],
    )(table, edge2, extra, zeros)


def kernel(nodes, edge_index, table):
    edge2 = edge_index.reshape(2, EB, BURST)
    npad = XTOT - (EREM + B)
    extra = jnp.concatenate(
        [edge_index[0, NS * DIRB * BURST:],
         jnp.arange(B, dtype=jnp.int32),
         jnp.full((npad,), PAD_DST, jnp.int32),
         edge_index[1, NS * DIRB * BURST:],
         nodes.astype(jnp.int32),
         jnp.zeros((npad,), jnp.int32)]).reshape(2 * NS, NEXT * CH, BURST)
    zeros = jnp.zeros((RPW, DH), jnp.float32)
    return _sc_aggregate(table, edge2, extra, zeros)


# speedup vs baseline: 2.7460x; 1.0192x over previous
import jax
import jax.numpy as jnp
from jax import lax
from jax.experimental import pallas as pl
from jax.experimental.pallas import tpu as pltpu
from jax.experimental.pallas import tpu_sc as plsc

NC = 2
NS = 16
L = 16
B = 10000
D = 128
DH = D // NC
E = 320000
BURST = 128
CH = 6
NDIR = 26
NEXT = 2
NCHUNK = NDIR + NEXT
NPAIR = NCHUNK // 2
NBURST = NCHUNK * CH
EB = E // BURST
DIRB = NDIR * CH
EREM = E - NS * DIRB * BURST
XTOT = NS * NEXT * CH * BURST
PAD_DST = B
ACC_ROWS = 10240
RPW = ACC_ROWS // NS
TPW = B // NS
NPIECE = 5
PJ = TPW // NPIECE
CW = 16
CROWS = ACC_ROWS // CW
CZPW = CROWS // NS


def _sc_body(table_h, edge_h, extra_h, zeros_h,
             out_h,
             srcC0, srcC1, dstC0, dstC1, rows0, rows1, res_v, cnt_v,
             hist_v, idxv,
             table_s, acc_s, cnt_s,
             g0, g1, s0, s1, i0, i1):
    c = lax.axis_index("c")
    s = lax.axis_index("s")

    SRC = (srcC0, srcC1)
    DST = (dstC0, dstC1)
    ROWS = (rows0, rows1)
    G = (g0, g1)
    S = (s0, s1)
    I = (i0, i1)

    ones16 = jnp.ones((L,), jnp.float32)

    init_cps = [
        (table_h.at[pl.ds(s * TPW, TPW), pl.ds(c * DH, DH)],
         table_s.at[pl.ds(s * TPW, TPW)], g0),
        (zeros_h, acc_s.at[pl.ds(s * RPW, RPW)], g1),
        (zeros_h.at[pl.ds(0, CZPW), pl.ds(0, CW)],
         cnt_s.at[pl.ds(s * CZPW, CZPW)], s0),
        (zeros_h.at[pl.ds(0, CROWS), pl.ds(0, CW)], hist_v, s1),
        (edge_h.at[1].at[pl.ds(s * DIRB, CH)], srcC0, i0),
        (edge_h.at[0].at[pl.ds(s * DIRB, CH)], dstC0, i1),
    ]
    for src_r, dst_r, sem in init_cps:
        pltpu.async_copy(src_r, dst_r, sem)
    for t in range(CROWS // L):
        idxv[pl.ds(t * L, L)] = lax.iota(jnp.int32, L) + (t * L)
    for src_r, dst_r, sem in init_cps:
        pltpu.make_async_copy(src_r, dst_r, sem).wait()
    plsc.subcore_barrier()

    def g_start(cp, j, bp):
        pltpu.async_copy(table_s.at[SRC[cp].at[j]], ROWS[bp], G[bp])

    def g_wait(cp, j, bp):
        pltpu.make_async_copy(table_s.at[SRC[cp].at[j]], ROWS[bp], G[bp]).wait()

    def s_start(cp, j, bp):
        pltpu.async_copy(ROWS[bp], acc_s.at[DST[cp].at[j]], S[bp], add=True)

    def s_wait(cp, j, bp):
        pltpu.make_async_copy(ROWS[bp], acc_s.at[DST[cp].at[j]], S[bp]).wait()

    def count(cp, j):
        for t in range(BURST // L):
            dv = DST[cp][j, pl.ds(t * L, L)]
            row = lax.shift_right_logical(dv, 4)
            lane = lax.bitwise_and(dv, 15)
            plsc.addupdate_scatter(hist_v, [row, lane], ones16)

    def pf_start_dir(ci, buf):
        off = s * DIRB + ci * CH
        pltpu.async_copy(edge_h.at[1].at[pl.ds(off, CH)], SRC[buf], I[buf])
        pltpu.async_copy(edge_h.at[0].at[pl.ds(off, CH)], DST[buf], I[buf])

    def pf_wait_dir(ci, buf):
        off = s * DIRB + ci * CH
        pltpu.make_async_copy(
            edge_h.at[1].at[pl.ds(off, CH)], SRC[buf], I[buf]).wait()
        pltpu.make_async_copy(
            edge_h.at[0].at[pl.ds(off, CH)], DST[buf], I[buf]).wait()

    def pf_start_ext(e, buf):
        pltpu.async_copy(extra_h.at[NS + s].at[pl.ds(e * CH, CH)],
                         SRC[buf], I[buf])
        pltpu.async_copy(extra_h.at[s].at[pl.ds(e * CH, CH)],
                         DST[buf], I[buf])

    def pf_wait_ext(e, buf):
        pltpu.make_async_copy(
            extra_h.at[NS + s].at[pl.ds(e * CH, CH)], SRC[buf], I[buf]).wait()
        pltpu.make_async_copy(
            extra_h.at[s].at[pl.ds(e * CH, CH)], DST[buf], I[buf]).wait()

    def do_chunk(p, pf_s, pf_w, first_chunk=False):
        for j in range(CH):
            bp = j % 2
            if first_chunk and j == 0:
                g_start(p, 0, 0)
            else:
                if j > 0:
                    s_wait(p, j - 1, 1 - bp)
                else:
                    s_wait(1 - p, CH - 1, 1 - bp)
            if j == 1:
                pf_s(1 - p)
            if j == CH - 1:
                pf_w(1 - p)
                g_start(1 - p, 0, 1 - bp)
            else:
                g_start(p, j + 1, 1 - bp)
            g_wait(p, j, bp)
            s_start(p, j, bp)
            count(p, j)

    def dirfns(ci):
        return (lambda buf: pf_start_dir(ci, buf),
                lambda buf: pf_wait_dir(ci, buf))

    def extfns(e):
        return (lambda buf: pf_start_ext(e, buf),
                lambda buf: pf_wait_ext(e, buf))

    do_chunk(0, *dirfns(1), first_chunk=True)
    do_chunk(1, *dirfns(2))

    @pl.loop(1, NPAIR - 2)
    def _(k):
        do_chunk(0, *dirfns(2 * k + 1))
        do_chunk(1, *dirfns(2 * k + 2))

    do_chunk(0, *dirfns(NDIR - 1))
    do_chunk(1, *extfns(0))
    do_chunk(0, *extfns(1))
    do_chunk(1, *extfns(0))

    g_wait(0, 0, 0)
    s_wait(1, CH - 1, 1)

    pltpu.sync_copy(hist_v, cnt_s.at[idxv], add=True)

    plsc.subcore_barrier()
    @pl.loop(0, NPIECE)
    def _(t):
        base = s * TPW + t * PJ
        crow0 = base // CW
        pltpu.sync_copy(acc_s.at[pl.ds(base, PJ)], res_v)
        pltpu.sync_copy(cnt_s.at[pl.ds(crow0, PJ // CW + 2)], cnt_v)

        @pl.loop(0, PJ)
        def _(r):
            flat = base + r
            rowp = flat // CW - crow0
            lanevec = jnp.broadcast_to(lax.rem(flat, CW), (L,))
            cnt = plsc.load_gather(cnt_v, [jnp.broadcast_to(rowp, (L,)),
                                           lanevec])
            inv = 1.0 / jnp.maximum(cnt, 1.0)
            for q in range(DH // L):
                res_v[r, pl.ds(q * L, L)] = res_v[r, pl.ds(q * L, L)] * inv

        pltpu.sync_copy(res_v,
                        out_h.at[pl.ds(base, PJ), pl.ds(c * DH, DH)])


def _sc_aggregate(table, edge2, extra, zeros):
    mesh = plsc.VectorSubcoreMesh(core_axis_name="c", subcore_axis_name="s")
    return pl.kernel(
        _sc_body,
        compiler_params=pltpu.CompilerParams(use_tc_tiling_on_sc=False,
                                             needs_layout_passes=False),
        out_type=jax.ShapeDtypeStruct((B, D), jnp.float32),
        mesh=mesh,
        scratch_types=[
            pltpu.VMEM((CH, BURST), jnp.int32),
            pltpu.VMEM((CH, BURST), jnp.int32),
            pltpu.VMEM((CH, BURST), jnp.int32),
            pltpu.VMEM((CH, BURST), jnp.int32),
            pltpu.VMEM((BURST, DH), jnp.float32),
            pltpu.VMEM((BURST, DH), jnp.float32),
            pltpu.VMEM((PJ, DH), jnp.float32),
            pltpu.VMEM((PJ // CW + 2, CW), jnp.float32),
            pltpu.VMEM((CROWS, CW), jnp.float32),
            pltpu.VMEM((CROWS,), jnp.int32),
            pltpu.VMEM_SHARED((B, DH), jnp.float32),
            pltpu.VMEM_SHARED((ACC_ROWS, DH), jnp.float32),
            pltpu.VMEM_SHARED((CROWS, CW), jnp.float32),
            pltpu.SemaphoreType.DMA,
            pltpu.SemaphoreType.DMA,
            pltpu.SemaphoreType.DMA,
            pltpu.SemaphoreType.DMA,
            pltpu.SemaphoreType.DMA,
            pltpu.SemaphoreType.DMA,
        ],
    )(table, edge2, extra, zeros)


def kernel(nodes, edge_index, table):
    edge2 = edge_index.reshape(2, EB, BURST)
    npad = XTOT - (EREM + B)
    extra = jnp.concatenate(
        [edge_index[0, NS * DIRB * BURST:],
         jnp.arange(B, dtype=jnp.int32),
         jnp.full((npad,), PAD_DST, jnp.int32),
         edge_index[1, NS * DIRB * BURST:],
         nodes.astype(jnp.int32),
         jnp.zeros((npad,), jnp.int32)]).reshape(2 * NS, NEXT * CH, BURST)
    zeros = jnp.zeros((RPW, DH), jnp.float32)
    return _sc_aggregate(table, edge2, extra, zeros)
